# baseline (speedup 1.0000x reference)
import functools

import jax
import jax.numpy as jnp
from jax import lax
from jax.experimental import pallas as pl
from jax.experimental.pallas import tpu as pltpu
from jax.experimental.pallas import tpu_sc as plsc

VOCAB = 1000000
EMBED_DIM = 16
BATCH = 16384
N_FIELDS = 26

_NW = 32
_B_PER_W = BATCH // _NW
_CB = 64
_NCHUNK = _B_PER_W // _CB
_CIDX = _CB * N_FIELDS


def _sig_kernel(table_hbm, xq_hbm, out_hbm, idx4_v, idx_v, rows_v, out_v, sem):
    wid = lax.axis_index("s") * 2 + lax.axis_index("c")
    for c in range(_NCHUNK):
        b0 = wid * _B_PER_W + c * _CB
        jt = wid * 4 + c // 2
        c0 = (c % 2) * _CB
        pltpu.sync_copy(xq_hbm.at[:, jt, :, pl.ds(c0, _CB)], idx4_v)
        for f in range(N_FIELDS):
            for k in range(_CB // 16):
                idx_v[pl.ds(f * _CB + k * 16, 16)] = \
                    idx4_v[f // 8, f % 8, pl.ds(k * 16, 16)]
        pltpu.async_copy(table_hbm.at[idx_v], rows_v, sem).wait()

        def body(bb, carry):
            for f in range(N_FIELDS):
                r = rows_v[f * _CB + bb]
                out_v[bb, f] = 1.0 / (1.0 + jnp.exp(-r))
            return carry

        lax.fori_loop(0, _CB, body, 0)
        pltpu.sync_copy(out_v, out_hbm.at[pl.ds(b0, _CB), :, :])


@jax.jit
def _run(table, xq):
    mesh = plsc.VectorSubcoreMesh(core_axis_name="c", subcore_axis_name="s")
    f = functools.partial(
        pl.kernel,
        mesh=mesh,
        out_type=jax.ShapeDtypeStruct((BATCH, N_FIELDS, EMBED_DIM), jnp.float32),
        scratch_types=[
            pltpu.VMEM((4, 8, _CB), jnp.int32),
            pltpu.VMEM((_CIDX,), jnp.int32),
            pltpu.VMEM((_CIDX, EMBED_DIM), jnp.float32),
            pltpu.VMEM((_CB, N_FIELDS, EMBED_DIM), jnp.float32),
            pltpu.SemaphoreType.DMA,
        ],
        compiler_params=pltpu.CompilerParams(use_tc_tiling_on_sc=False),
    )(_sig_kernel)
    return f(table, xq)


def kernel(table, x):
    xp = jnp.pad(x, ((0, 0), (0, 32 - N_FIELDS)))
    xq = xp.T.reshape(4, 8, 128, 128).transpose(0, 2, 1, 3)
    return _run(table, xq)


# speedup vs baseline: 1.2983x; 1.2983x over previous
import functools

import jax
import jax.numpy as jnp
from jax import lax
from jax.experimental import pallas as pl
from jax.experimental.pallas import tpu as pltpu
from jax.experimental.pallas import tpu_sc as plsc

VOCAB = 1000000
EMBED_DIM = 16
BATCH = 16384
N_FIELDS = 26

_NW = 32
_B_PER_W = BATCH // _NW
_CB = 64
_NCHUNK = _B_PER_W // _CB


def _sig_kernel(table_hbm, x_hbm, out_hbm, idx2_v, idx_v, rows_v, out_v, sem):
    wid = lax.axis_index("s") * 2 + lax.axis_index("c")
    base = wid * _B_PER_W
    for c in range(_NCHUNK):
        b0 = base + c * _CB
        pltpu.sync_copy(x_hbm.at[pl.ds(b0, _CB), :], idx2_v)

        def repack(bb, carry):
            a = idx2_v[bb, pl.ds(0, 16)]
            b = idx2_v[bb, pl.ds(N_FIELDS - 16, 16)]
            idx_v[pl.ds(bb * N_FIELDS, 16)] = a
            idx_v[pl.ds(bb * N_FIELDS + N_FIELDS - 16, 16)] = b
            return carry

        lax.fori_loop(0, _CB, repack, 0)
        pltpu.async_copy(table_hbm.at[idx_v], rows_v, sem).wait()

        def body(bb, carry):
            j0 = bb * N_FIELDS
            for f in range(N_FIELDS):
                r = rows_v[j0 + f]
                out_v[bb, f] = 1.0 / (1.0 + jnp.exp(-r))
            return carry

        lax.fori_loop(0, _CB, body, 0)
        pltpu.sync_copy(out_v, out_hbm.at[pl.ds(b0, _CB), :, :])


@jax.jit
def _run(table, x):
    mesh = plsc.VectorSubcoreMesh(core_axis_name="c", subcore_axis_name="s")
    f = functools.partial(
        pl.kernel,
        mesh=mesh,
        out_type=jax.ShapeDtypeStruct((BATCH, N_FIELDS, EMBED_DIM), jnp.float32),
        scratch_types=[
            pltpu.VMEM((_CB, N_FIELDS), jnp.int32),
            pltpu.VMEM((_CB * N_FIELDS,), jnp.int32),
            pltpu.VMEM((_CB * N_FIELDS, EMBED_DIM), jnp.float32),
            pltpu.VMEM((_CB, N_FIELDS, EMBED_DIM), jnp.float32),
            pltpu.SemaphoreType.DMA,
        ],
        compiler_params=pltpu.CompilerParams(use_tc_tiling_on_sc=False),
    )(_sig_kernel)
    return f(table, x)


def kernel(table, x):
    return _run(table, x)
